# baseline (speedup 1.0000x reference)
import jax
import jax.numpy as jnp
from jax import lax
from jax.experimental import pallas as pl
from jax.experimental.pallas import tpu as pltpu
from jax.experimental.pallas import tpu_sc as plsc

_K = 8
_NK = 64
_H = 4
_T = 2048
_DM = 1024
_DCD = 2048
_DPE = 512
_NE = 4096
_G = _NE // 128
_BT = 256
_NC = 2
_NS = 16
_NW = _NC * _NS
_TPW = _T // _NW
_C = 8
_NEG = float("-inf")


def _front_body(x_ref, wup_ref, bup_ref, wdown_ref, bdown_ref, wq_ref,
                kmt_ref, det_ref, logits_ref, gidx_ref, ss_ref):
    pid = pl.program_id(0)
    x = x_ref[...]
    h1 = jnp.dot(x, wup_ref[...], preferred_element_type=jnp.float32)
    h1 = h1 + bup_ref[...]
    h1 = h1 * (1.0 / (1.0 + jnp.exp(-h1)))
    h = jnp.dot(h1, wdown_ref[...], preferred_element_type=jnp.float32)
    h = h + bdown_ref[...]
    lb = jnp.dot(h.astype(jnp.bfloat16), det_ref[...],
                 preferred_element_type=jnp.float32)
    for g in range(_G):
        logits_ref[pl.ds(g * _BT * 128, _BT * 128)] = (
            lb[:, g * 128:(g + 1) * 128].reshape(_BT * 128))
    q = jnp.dot(h, wq_ref[...], preferred_element_type=jnp.float32)
    simt = lax.dot_general(kmt_ref[...], q, (((1,), (1,)), ((), ())),
                           preferred_element_type=jnp.float32)

    riota = lax.broadcasted_iota(jnp.int32, (_NK, _BT), 0)
    tloc = lax.broadcasted_iota(jnp.int32, (1, _BT), 1)
    row32 = lax.broadcasted_iota(jnp.int32, (4 * _K, _BT), 0)

    def top8(s):
        vals, poss = [], []
        for _ in range(_K):
            m = jnp.max(s, axis=0, keepdims=True)
            am = jnp.min(jnp.where(s == m, riota, _NK), axis=0, keepdims=True)
            vals.append(m)
            poss.append(am)
            s = jnp.where(riota == am, _NEG, s)
        return vals, poss

    out_ss = jnp.zeros((4 * _K, _BT), jnp.float32)
    out_gi = jnp.zeros((4 * _K, _BT), jnp.int32)
    ra = riota // _K
    rb = riota % _K
    for hh in range(_H):
        xv, xp = top8(simt[hh * _NK:(hh + 1) * _NK, :])
        yv, yp = top8(simt[_H * _NK + hh * _NK:_H * _NK + (hh + 1) * _NK, :])
        asc = jnp.zeros((_NK, _BT), jnp.float32)
        aidx = jnp.zeros((_NK, _BT), jnp.int32)
        for a in range(_K):
            asc = asc + jnp.where(ra == a, xv[a], 0.0)
            aidx = aidx + jnp.where(ra == a, xp[a] * _NK, 0)
        for b in range(_K):
            asc = asc + jnp.where(rb == b, yv[b], 0.0)
            aidx = aidx + jnp.where(rb == b, yp[b], 0)
        scs, eids = [], []
        for _ in range(_K):
            m = jnp.max(asc, axis=0, keepdims=True)
            am = jnp.min(jnp.where(asc == m, riota, _NK), axis=0, keepdims=True)
            sel = riota == am
            eids.append(jnp.sum(jnp.where(sel, aidx, 0), axis=0, keepdims=True))
            scs.append(m)
            asc = jnp.where(sel, _NEG, asc)
        es = [jnp.exp(v - scs[0]) for v in scs]
        tot = es[0]
        for e in es[1:]:
            tot = tot + e
        inv = 1.0 / tot
        for k in range(_K):
            r = hh * _K + k
            gf = (pid * (_G * _BT * 128) + (eids[k] // 128) * (_BT * 128)
                  + tloc * 128 + (eids[k] % 128))
            out_ss = jnp.where(row32 == r, es[k] * inv, out_ss)
            out_gi = jnp.where(row32 == r, gf, out_gi)
    ss_ref[...] = jnp.transpose(out_ss)
    gidx_ref[...] = jnp.transpose(out_gi)


def _combine_body(wsp_ref, ue_ref, out_ref):
    accs = [None] * 4
    for g in range(_G):
        d = jnp.dot(wsp_ref[g].astype(jnp.bfloat16), ue_ref[g],
                    preferred_element_type=jnp.float32)
        a = g & 3
        accs[a] = d if accs[a] is None else accs[a] + d
    out_ref[...] = (accs[0] + accs[1]) + (accs[2] + accs[3])


def _sc_body(logits_hbm, gidx_hbm, ss_hbm, zc_hbm, wsp_hbm, gi0_v, gi1_v,
             ss_v, xg_v, chunk_v, gsem, dsem0, dsem1):
    cid = lax.axis_index("c")
    sid = lax.axis_index("s")
    wid = sid * _NC + cid
    base = wid * _TPW
    nsel = 4 * _K * _C

    zero16 = jnp.zeros((16,), jnp.float32)
    pltpu.sync_copy(zc_hbm, chunk_v.at[0])
    pltpu.sync_copy(zc_hbm, chunk_v.at[1])

    dsems = [dsem0, dsem1]
    gis = [gi0_v, gi1_v]
    out_dma = [None, None]
    prev = [None, None]
    for ci in range(_TPW // _C):
        b = ci & 1
        t0 = base + ci * _C
        tb = t0 & (_BT - 1)
        if out_dma[b] is not None:
            out_dma[b].wait()
            ptb = prev[b]
            for j in range(nsel // 16):
                gf = gis[b][pl.ds(j * 16, 16)]
                plsc.store_scatter(
                    chunk_v.at[b],
                    [lax.shift_right_logical(gf, 15) & (_G - 1),
                     (lax.shift_right_logical(gf, 7) & (_BT - 1)) - ptb,
                     gf & 127],
                    zero16)
        pltpu.sync_copy(gidx_hbm.at[pl.ds(t0 * 4 * _K, nsel)], gis[b])
        pltpu.sync_copy(ss_hbm.at[pl.ds(t0 * 4 * _K, nsel)], ss_v)
        pltpu.async_copy(logits_hbm.at[gis[b]], xg_v, gsem).wait()
        for j in range(nsel // 16):
            sl = pl.ds(j * 16, 16)
            z = xg_v[sl] * ss_v[sl]
            w = z * (1.0 / (1.0 + jnp.exp(-z)))
            gf = gis[b][sl]
            plsc.addupdate_scatter(
                chunk_v.at[b],
                [lax.shift_right_logical(gf, 15) & (_G - 1),
                 (lax.shift_right_logical(gf, 7) & (_BT - 1)) - tb,
                 gf & 127],
                w)
        out_dma[b] = pltpu.async_copy(chunk_v.at[b],
                                      wsp_hbm.at[:, pl.ds(t0, _C), :],
                                      dsems[b])
        prev[b] = tb
    out_dma[0].wait()
    out_dma[1].wait()


def _routing_sc(logits, gidx, ss):
    mesh = plsc.VectorSubcoreMesh(core_axis_name="c", subcore_axis_name="s")
    f = pl.kernel(
        _sc_body,
        out_type=jax.ShapeDtypeStruct((_G, _T, 128), jnp.float32),
        mesh=mesh,
        scratch_types=[
            pltpu.VMEM((4 * _K * _C,), jnp.int32),
            pltpu.VMEM((4 * _K * _C,), jnp.int32),
            pltpu.VMEM((4 * _K * _C,), jnp.float32),
            pltpu.VMEM((4 * _K * _C,), jnp.float32),
            pltpu.VMEM((2, _G, _C, 128), jnp.float32),
            pltpu.SemaphoreType.DMA,
            pltpu.SemaphoreType.DMA,
            pltpu.SemaphoreType.DMA,
        ],
        compiler_params=pltpu.CompilerParams(needs_layout_passes=False),
    )
    return f(logits, gidx.reshape(_T * 4 * _K), ss.reshape(_T * 4 * _K),
             jnp.zeros((_G, _C, 128), jnp.float32))


def kernel(hidden_states, W_up, b_up, W_down, b_down, W_q, keys, down_embed,
           up_embed):
    x = hidden_states.reshape(_T, _DM)
    kk = keys.transpose(2, 0, 1, 3).reshape(2 * _H, _NK, _NK)
    eye8 = jnp.eye(2 * _H, dtype=keys.dtype)
    kmt = jnp.einsum('gkn,gG->gkGn', kk, eye8).reshape(2 * _H * _NK,
                                                       2 * _H * _NK)

    grid = _T // _BT
    logits, gidx, ss = pl.pallas_call(
        _front_body,
        grid=(grid,),
        in_specs=[
            pl.BlockSpec((_BT, _DM), lambda i: (i, 0)),
            pl.BlockSpec((_DM, _DCD), lambda i: (0, 0)),
            pl.BlockSpec((1, _DCD), lambda i: (0, 0)),
            pl.BlockSpec((_DCD, _DPE), lambda i: (0, 0)),
            pl.BlockSpec((1, _DPE), lambda i: (0, 0)),
            pl.BlockSpec((_DPE, _DPE), lambda i: (0, 0)),
            pl.BlockSpec((_DPE, _DPE), lambda i: (0, 0)),
            pl.BlockSpec((_DPE, _NE), lambda i: (0, 0)),
        ],
        out_specs=[
            pl.BlockSpec((_G * _BT * 128,), lambda i: (i,)),
            pl.BlockSpec((_BT, 4 * _K), lambda i: (i, 0)),
            pl.BlockSpec((_BT, 4 * _K), lambda i: (i, 0)),
        ],
        out_shape=[
            jax.ShapeDtypeStruct(((_T // _BT) * _G * _BT * 128,), jnp.float32),
            jax.ShapeDtypeStruct((_T, 4 * _K), jnp.int32),
            jax.ShapeDtypeStruct((_T, 4 * _K), jnp.float32),
        ],
    )(x, W_up, b_up.reshape(1, _DCD), W_down, b_down.reshape(1, _DPE), W_q,
      kmt, down_embed.T.astype(jnp.bfloat16))

    wsp = _routing_sc(logits, gidx, ss)

    ueb = up_embed.astype(jnp.bfloat16).reshape(_G, 128, _DM)
    out = pl.pallas_call(
        _combine_body,
        grid=(grid,),
        in_specs=[
            pl.BlockSpec((_G, _BT, 128), lambda i: (0, i, 0)),
            pl.BlockSpec((_G, 128, _DM), lambda i: (0, 0, 0)),
        ],
        out_specs=pl.BlockSpec((_BT, _DM), lambda i: (i, 0)),
        out_shape=jax.ShapeDtypeStruct((_T, _DM), jnp.float32),
    )(wsp, ueb)

    return out.reshape(1, _T, _DM)


# speedup vs baseline: 1.0400x; 1.0400x over previous
import jax
import jax.numpy as jnp
from jax import lax
from jax.experimental import pallas as pl
from jax.experimental.pallas import tpu as pltpu
from jax.experimental.pallas import tpu_sc as plsc

_K = 8
_NK = 64
_H = 4
_T = 2048
_DM = 1024
_DCD = 2048
_DPE = 512
_NE = 4096
_G = _NE // 128
_BT = 256
_NC = 2
_NS = 16
_NW = _NC * _NS
_TPW = _T // _NW
_C = 16
_NEG = float("-inf")


def _front_body(x_ref, wup_ref, bup_ref, wdown_ref, bdown_ref, wq_ref,
                kmt_ref, det_ref, logits_ref, gidx_ref, ss_ref):
    pid = pl.program_id(0)
    x = x_ref[...]
    h1 = jnp.dot(x, wup_ref[...], preferred_element_type=jnp.float32)
    h1 = h1 + bup_ref[...]
    h1 = h1 * (1.0 / (1.0 + jnp.exp(-h1)))
    h = jnp.dot(h1, wdown_ref[...], preferred_element_type=jnp.float32)
    h = h + bdown_ref[...]
    lb = jnp.dot(h.astype(jnp.bfloat16), det_ref[...],
                 preferred_element_type=jnp.float32)
    for g in range(_G):
        logits_ref[pl.ds(g * _BT * 128, _BT * 128)] = (
            lb[:, g * 128:(g + 1) * 128].reshape(_BT * 128))
    q = jnp.dot(h, wq_ref[...], preferred_element_type=jnp.float32)
    simt = lax.dot_general(kmt_ref[...], q, (((1,), (1,)), ((), ())),
                           preferred_element_type=jnp.float32)

    riota = lax.broadcasted_iota(jnp.int32, (_NK, _BT), 0)
    tloc = lax.broadcasted_iota(jnp.int32, (1, _BT), 1)
    row32 = lax.broadcasted_iota(jnp.int32, (4 * _K, _BT), 0)

    def top8(s):
        vals, poss = [], []
        for _ in range(_K):
            m = jnp.max(s, axis=0, keepdims=True)
            am = jnp.min(jnp.where(s == m, riota, _NK), axis=0, keepdims=True)
            vals.append(m)
            poss.append(am)
            s = jnp.where(riota == am, _NEG, s)
        return vals, poss

    out_ss = jnp.zeros((4 * _K, _BT), jnp.float32)
    out_gi = jnp.zeros((4 * _K, _BT), jnp.int32)
    ra = riota // _K
    rb = riota % _K
    for hh in range(_H):
        xv, xp = top8(simt[hh * _NK:(hh + 1) * _NK, :])
        yv, yp = top8(simt[_H * _NK + hh * _NK:_H * _NK + (hh + 1) * _NK, :])
        asc = jnp.zeros((_NK, _BT), jnp.float32)
        aidx = jnp.zeros((_NK, _BT), jnp.int32)
        for a in range(_K):
            asc = asc + jnp.where(ra == a, xv[a], 0.0)
            aidx = aidx + jnp.where(ra == a, xp[a] * _NK, 0)
        for b in range(_K):
            asc = asc + jnp.where(rb == b, yv[b], 0.0)
            aidx = aidx + jnp.where(rb == b, yp[b], 0)
        scs, eids = [], []
        for _ in range(_K):
            m = jnp.max(asc, axis=0, keepdims=True)
            am = jnp.min(jnp.where(asc == m, riota, _NK), axis=0, keepdims=True)
            sel = riota == am
            eids.append(jnp.sum(jnp.where(sel, aidx, 0), axis=0, keepdims=True))
            scs.append(m)
            asc = jnp.where(sel, _NEG, asc)
        es = [jnp.exp(v - scs[0]) for v in scs]
        tot = es[0]
        for e in es[1:]:
            tot = tot + e
        inv = 1.0 / tot
        for k in range(_K):
            r = hh * _K + k
            gf = (pid * (_G * _BT * 128) + (eids[k] // 128) * (_BT * 128)
                  + tloc * 128 + (eids[k] % 128))
            out_ss = jnp.where(row32 == r, es[k] * inv, out_ss)
            out_gi = jnp.where(row32 == r, gf, out_gi)
    ss_ref[...] = jnp.transpose(out_ss)
    gidx_ref[...] = jnp.transpose(out_gi)


def _combine_body(wsp_ref, ue_ref, out_ref):
    accs = [None] * 4
    for g in range(_G):
        d = jnp.dot(wsp_ref[g].astype(jnp.bfloat16), ue_ref[g],
                    preferred_element_type=jnp.float32)
        a = g & 3
        accs[a] = d if accs[a] is None else accs[a] + d
    out_ref[...] = (accs[0] + accs[1]) + (accs[2] + accs[3])


def _sc_body(logits_hbm, gidx_hbm, ss_hbm, zc_hbm, wsp_hbm, gi0_v, gi1_v,
             ss_v, xg_v, wv_v, chunk_v, gsem, dsem0):
    cid = lax.axis_index("c")
    sid = lax.axis_index("s")
    wid = sid * _NC + cid
    base = wid * _TPW
    nsel = 4 * _K * _C

    zero16 = jnp.zeros((16,), jnp.float32)
    pltpu.sync_copy(zc_hbm, chunk_v)

    gis = [gi0_v, gi1_v]
    out_dma = None
    ptb = None
    for ci in range(_TPW // _C):
        b = ci & 1
        t0 = base + ci * _C
        tb = t0 & (_BT - 1)
        pltpu.sync_copy(gidx_hbm.at[pl.ds(t0 * 4 * _K, nsel)], gis[b])
        pltpu.sync_copy(ss_hbm.at[pl.ds(t0 * 4 * _K, nsel)], ss_v)
        pltpu.async_copy(logits_hbm.at[gis[b]], xg_v, gsem).wait()
        for j in range(nsel // 16):
            sl = pl.ds(j * 16, 16)
            z = xg_v[sl] * ss_v[sl]
            wv_v[sl] = z * (1.0 / (1.0 + jnp.exp(-z)))
        if out_dma is not None:
            out_dma.wait()
            gp = gis[1 - b]
            for j in range(nsel // 16):
                gf = gp[pl.ds(j * 16, 16)]
                plsc.store_scatter(
                    chunk_v,
                    [lax.shift_right_logical(gf, 15) & (_G - 1),
                     (lax.shift_right_logical(gf, 7) & (_BT - 1)) - ptb,
                     gf & 127],
                    zero16)
        for j in range(nsel // 16):
            sl = pl.ds(j * 16, 16)
            gf = gis[b][sl]
            plsc.addupdate_scatter(
                chunk_v,
                [lax.shift_right_logical(gf, 15) & (_G - 1),
                 (lax.shift_right_logical(gf, 7) & (_BT - 1)) - tb,
                 gf & 127],
                wv_v[sl])
        out_dma = pltpu.async_copy(chunk_v, wsp_hbm.at[:, pl.ds(t0, _C), :],
                                   dsem0)
        ptb = tb
    out_dma.wait()


def _routing_sc(logits, gidx, ss):
    mesh = plsc.VectorSubcoreMesh(core_axis_name="c", subcore_axis_name="s")
    f = pl.kernel(
        _sc_body,
        out_type=jax.ShapeDtypeStruct((_G, _T, 128), jnp.float32),
        mesh=mesh,
        scratch_types=[
            pltpu.VMEM((4 * _K * _C,), jnp.int32),
            pltpu.VMEM((4 * _K * _C,), jnp.int32),
            pltpu.VMEM((4 * _K * _C,), jnp.float32),
            pltpu.VMEM((4 * _K * _C,), jnp.float32),
            pltpu.VMEM((4 * _K * _C,), jnp.float32),
            pltpu.VMEM((_G, _C, 128), jnp.float32),
            pltpu.SemaphoreType.DMA,
            pltpu.SemaphoreType.DMA,
        ],
        compiler_params=pltpu.CompilerParams(needs_layout_passes=False),
    )
    return f(logits, gidx.reshape(_T * 4 * _K), ss.reshape(_T * 4 * _K),
             jnp.zeros((_G, _C, 128), jnp.float32))


def kernel(hidden_states, W_up, b_up, W_down, b_down, W_q, keys, down_embed,
           up_embed):
    x = hidden_states.reshape(_T, _DM)
    kk = keys.transpose(2, 0, 1, 3).reshape(2 * _H, _NK, _NK)
    eye8 = jnp.eye(2 * _H, dtype=keys.dtype)
    kmt = jnp.einsum('gkn,gG->gkGn', kk, eye8).reshape(2 * _H * _NK,
                                                       2 * _H * _NK)

    grid = _T // _BT
    logits, gidx, ss = pl.pallas_call(
        _front_body,
        grid=(grid,),
        in_specs=[
            pl.BlockSpec((_BT, _DM), lambda i: (i, 0)),
            pl.BlockSpec((_DM, _DCD), lambda i: (0, 0)),
            pl.BlockSpec((1, _DCD), lambda i: (0, 0)),
            pl.BlockSpec((_DCD, _DPE), lambda i: (0, 0)),
            pl.BlockSpec((1, _DPE), lambda i: (0, 0)),
            pl.BlockSpec((_DPE, _DPE), lambda i: (0, 0)),
            pl.BlockSpec((_DPE, _DPE), lambda i: (0, 0)),
            pl.BlockSpec((_DPE, _NE), lambda i: (0, 0)),
        ],
        out_specs=[
            pl.BlockSpec((_G * _BT * 128,), lambda i: (i,)),
            pl.BlockSpec((_BT, 4 * _K), lambda i: (i, 0)),
            pl.BlockSpec((_BT, 4 * _K), lambda i: (i, 0)),
        ],
        out_shape=[
            jax.ShapeDtypeStruct(((_T // _BT) * _G * _BT * 128,), jnp.float32),
            jax.ShapeDtypeStruct((_T, 4 * _K), jnp.int32),
            jax.ShapeDtypeStruct((_T, 4 * _K), jnp.float32),
        ],
    )(x, W_up, b_up.reshape(1, _DCD), W_down, b_down.reshape(1, _DPE), W_q,
      kmt, down_embed.T.astype(jnp.bfloat16))

    wsp = _routing_sc(logits, gidx, ss)

    ueb = up_embed.astype(jnp.bfloat16).reshape(_G, 128, _DM)
    out = pl.pallas_call(
        _combine_body,
        grid=(grid,),
        in_specs=[
            pl.BlockSpec((_G, _BT, 128), lambda i: (0, i, 0)),
            pl.BlockSpec((_G, 128, _DM), lambda i: (0, 0, 0)),
        ],
        out_specs=pl.BlockSpec((_BT, _DM), lambda i: (i, 0)),
        out_shape=jax.ShapeDtypeStruct((_T, _DM), jnp.float32),
    )(wsp, ueb)

    return out.reshape(1, _T, _DM)


# speedup vs baseline: 1.0501x; 1.0097x over previous
import jax
import jax.numpy as jnp
from jax import lax
from jax.experimental import pallas as pl
from jax.experimental.pallas import tpu as pltpu
from jax.experimental.pallas import tpu_sc as plsc

_K = 8
_NK = 64
_H = 4
_T = 2048
_DM = 1024
_DCD = 2048
_DPE = 512
_NE = 4096
_G = _NE // 128
_BT = 256
_NC = 2
_NS = 16
_NW = _NC * _NS
_TPW = _T // _NW
_C = 16
_NEG = float("-inf")


_CAND = [(a, b) for a in range(_K) for b in range(_K)
         if (a + 1) * (b + 1) <= _K]
_NCAND = len(_CAND)


def _front_body(x_ref, wup_ref, bup_ref, wdown_ref, bdown_ref, wq_ref,
                kk_ref, det_ref, logits_ref, gidx_ref, ss_ref):
    pid = pl.program_id(0)
    x = x_ref[...]
    h1 = jnp.dot(x, wup_ref[...], preferred_element_type=jnp.float32)
    h1 = h1 + bup_ref[...]
    h1 = h1 * (1.0 / (1.0 + jnp.exp(-h1)))
    h = jnp.dot(h1, wdown_ref[...], preferred_element_type=jnp.float32)
    h = h + bdown_ref[...]
    lb = lax.dot_general(h.astype(jnp.bfloat16), det_ref[...],
                         (((1,), (1,)), ((), ())),
                         preferred_element_type=jnp.float32)
    for g in range(_G):
        logits_ref[pl.ds(g * _BT * 128, _BT * 128)] = (
            lb[:, g * 128:(g + 1) * 128].reshape(_BT * 128))

    simts = []
    for g in range(2 * _H):
        qg = jnp.dot(h, wq_ref[:, g * _NK:(g + 1) * _NK],
                     preferred_element_type=jnp.float32)
        simts.append(lax.dot_general(kk_ref[g], qg, (((1,), (1,)), ((), ())),
                                     preferred_element_type=jnp.float32))

    riota = lax.broadcasted_iota(jnp.int32, (_NK, _BT), 0)
    ciota = lax.broadcasted_iota(jnp.int32, (_NCAND, _BT), 0)
    tloc = lax.broadcasted_iota(jnp.int32, (1, _BT), 1)
    row32 = lax.broadcasted_iota(jnp.int32, (4 * _K, _BT), 0)

    def top8(s):
        vals, poss = [], []
        for _ in range(_K):
            m = jnp.max(s, axis=0, keepdims=True)
            am = jnp.min(jnp.where(s == m, riota, _NK), axis=0, keepdims=True)
            vals.append(m)
            poss.append(am)
            s = jnp.where(riota == am, _NEG, s)
        return vals, poss

    out_ss = jnp.zeros((4 * _K, _BT), jnp.float32)
    out_gi = jnp.zeros((4 * _K, _BT), jnp.int32)
    for hh in range(_H):
        xv, xp = top8(simts[hh])
        yv, yp = top8(simts[_H + hh])
        asc = jnp.concatenate([xv[a] + yv[b] for a, b in _CAND], axis=0)
        aidx = jnp.concatenate([xp[a] * _NK + yp[b] for a, b in _CAND],
                               axis=0)
        scs, eids = [], []
        for _ in range(_K):
            m = jnp.max(asc, axis=0, keepdims=True)
            am = jnp.min(jnp.where(asc == m, ciota, _NCAND), axis=0,
                         keepdims=True)
            sel = ciota == am
            eids.append(jnp.sum(jnp.where(sel, aidx, 0), axis=0, keepdims=True))
            scs.append(m)
            asc = jnp.where(sel, _NEG, asc)
        es = [jnp.exp(v - scs[0]) for v in scs]
        tot = es[0]
        for e in es[1:]:
            tot = tot + e
        inv = 1.0 / tot
        for k in range(_K):
            r = hh * _K + k
            gf = (pid * (_G * _BT * 128) + (eids[k] // 128) * (_BT * 128)
                  + tloc * 128 + (eids[k] % 128))
            out_ss = jnp.where(row32 == r, es[k] * inv, out_ss)
            out_gi = jnp.where(row32 == r, gf, out_gi)
    ss_ref[...] = jnp.transpose(out_ss)
    gidx_ref[...] = jnp.transpose(out_gi)


def _combine_body(wsp_ref, ue_ref, out_ref):
    accs = [None] * 4
    for g in range(_G):
        d = jnp.dot(wsp_ref[g].astype(jnp.bfloat16), ue_ref[g],
                    preferred_element_type=jnp.float32)
        a = g & 3
        accs[a] = d if accs[a] is None else accs[a] + d
    out_ref[...] = (accs[0] + accs[1]) + (accs[2] + accs[3])


def _sc_body(logits_hbm, gidx_hbm, ss_hbm, zc_hbm, wsp_hbm, gi0_v, gi1_v,
             ss_v, xg_v, wv_v, chunk_v, gsem, dsem0):
    cid = lax.axis_index("c")
    sid = lax.axis_index("s")
    wid = sid * _NC + cid
    base = wid * _TPW
    nsel = 4 * _K * _C

    zero16 = jnp.zeros((16,), jnp.float32)
    pltpu.sync_copy(zc_hbm, chunk_v)

    gis = [gi0_v, gi1_v]
    out_dma = None
    ptb = None
    for ci in range(_TPW // _C):
        b = ci & 1
        t0 = base + ci * _C
        tb = t0 & (_BT - 1)
        pltpu.sync_copy(gidx_hbm.at[pl.ds(t0 * 4 * _K, nsel)], gis[b])
        pltpu.sync_copy(ss_hbm.at[pl.ds(t0 * 4 * _K, nsel)], ss_v)
        pltpu.async_copy(logits_hbm.at[gis[b]], xg_v, gsem).wait()
        for j in range(nsel // 16):
            sl = pl.ds(j * 16, 16)
            z = xg_v[sl] * ss_v[sl]
            wv_v[sl] = z * (1.0 / (1.0 + jnp.exp(-z)))
        if out_dma is not None:
            out_dma.wait()
            gp = gis[1 - b]
            for j in range(nsel // 16):
                gf = gp[pl.ds(j * 16, 16)]
                plsc.store_scatter(
                    chunk_v,
                    [lax.shift_right_logical(gf, 15) & (_G - 1),
                     (lax.shift_right_logical(gf, 7) & (_BT - 1)) - ptb,
                     gf & 127],
                    zero16)
        for j in range(nsel // 16):
            sl = pl.ds(j * 16, 16)
            gf = gis[b][sl]
            plsc.addupdate_scatter(
                chunk_v,
                [lax.shift_right_logical(gf, 15) & (_G - 1),
                 (lax.shift_right_logical(gf, 7) & (_BT - 1)) - tb,
                 gf & 127],
                wv_v[sl])
        out_dma = pltpu.async_copy(chunk_v, wsp_hbm.at[:, pl.ds(t0, _C), :],
                                   dsem0)
        ptb = tb
    out_dma.wait()


def _routing_sc(logits, gidx, ss):
    mesh = plsc.VectorSubcoreMesh(core_axis_name="c", subcore_axis_name="s")
    f = pl.kernel(
        _sc_body,
        out_type=jax.ShapeDtypeStruct((_G, _T, 128), jnp.float32),
        mesh=mesh,
        scratch_types=[
            pltpu.VMEM((4 * _K * _C,), jnp.int32),
            pltpu.VMEM((4 * _K * _C,), jnp.int32),
            pltpu.VMEM((4 * _K * _C,), jnp.float32),
            pltpu.VMEM((4 * _K * _C,), jnp.float32),
            pltpu.VMEM((4 * _K * _C,), jnp.float32),
            pltpu.VMEM((_G, _C, 128), jnp.float32),
            pltpu.SemaphoreType.DMA,
            pltpu.SemaphoreType.DMA,
        ],
        compiler_params=pltpu.CompilerParams(needs_layout_passes=False),
    )
    return f(logits, gidx.reshape(_T * 4 * _K), ss.reshape(_T * 4 * _K),
             jnp.zeros((_G, _C, 128), jnp.float32))


def kernel(hidden_states, W_up, b_up, W_down, b_down, W_q, keys, down_embed,
           up_embed):
    x = hidden_states.reshape(_T, _DM)
    kk = keys.transpose(2, 0, 1, 3).reshape(2 * _H, _NK, _NK)

    grid = _T // _BT
    logits, gidx, ss = pl.pallas_call(
        _front_body,
        grid=(grid,),
        in_specs=[
            pl.BlockSpec((_BT, _DM), lambda i: (i, 0)),
            pl.BlockSpec((_DM, _DCD), lambda i: (0, 0)),
            pl.BlockSpec((1, _DCD), lambda i: (0, 0)),
            pl.BlockSpec((_DCD, _DPE), lambda i: (0, 0)),
            pl.BlockSpec((1, _DPE), lambda i: (0, 0)),
            pl.BlockSpec((_DPE, _DPE), lambda i: (0, 0)),
            pl.BlockSpec((2 * _H, _NK, _NK), lambda i: (0, 0, 0)),
            pl.BlockSpec((_NE, _DPE), lambda i: (0, 0)),
        ],
        out_specs=[
            pl.BlockSpec((_G * _BT * 128,), lambda i: (i,)),
            pl.BlockSpec((_BT, 4 * _K), lambda i: (i, 0)),
            pl.BlockSpec((_BT, 4 * _K), lambda i: (i, 0)),
        ],
        out_shape=[
            jax.ShapeDtypeStruct(((_T // _BT) * _G * _BT * 128,), jnp.float32),
            jax.ShapeDtypeStruct((_T, 4 * _K), jnp.int32),
            jax.ShapeDtypeStruct((_T, 4 * _K), jnp.float32),
        ],
    )(x, W_up, b_up.reshape(1, _DCD), W_down, b_down.reshape(1, _DPE), W_q,
      kk, down_embed.astype(jnp.bfloat16))

    wsp = _routing_sc(logits, gidx, ss)

    ueb = up_embed.astype(jnp.bfloat16).reshape(_G, 128, _DM)
    out = pl.pallas_call(
        _combine_body,
        grid=(grid,),
        in_specs=[
            pl.BlockSpec((_G, _BT, 128), lambda i: (0, i, 0)),
            pl.BlockSpec((_G, 128, _DM), lambda i: (0, 0, 0)),
        ],
        out_specs=pl.BlockSpec((_BT, _DM), lambda i: (i, 0)),
        out_shape=jax.ShapeDtypeStruct((_T, _DM), jnp.float32),
    )(wsp, ueb)

    return out.reshape(1, _T, _DM)


# speedup vs baseline: 1.0502x; 1.0001x over previous
import jax
import jax.numpy as jnp
from jax import lax
from jax.experimental import pallas as pl
from jax.experimental.pallas import tpu as pltpu
from jax.experimental.pallas import tpu_sc as plsc

_K = 8
_NK = 64
_H = 4
_T = 2048
_DM = 1024
_DCD = 2048
_DPE = 512
_NE = 4096
_G = _NE // 128
_BT = 256
_NC = 2
_NS = 16
_NW = _NC * _NS
_TPW = _T // _NW
_C = 16
_NEG = float("-inf")


_CAND = [(a, b) for a in range(_K) for b in range(_K)
         if (a + 1) * (b + 1) <= _K]
_NCAND = len(_CAND)


def _front_body(x_ref, wup_ref, bup_ref, wdown_ref, bdown_ref, wq_ref,
                kk_ref, det_ref, logits_ref, gidx_ref, ss_ref):
    pid = pl.program_id(0)
    x = x_ref[...]
    h1 = jnp.dot(x, wup_ref[...], preferred_element_type=jnp.float32)
    h1 = h1 + bup_ref[...]
    h1 = h1 * (1.0 / (1.0 + jnp.exp(-h1)))
    h = jnp.dot(h1, wdown_ref[...], preferred_element_type=jnp.float32)
    h = h + bdown_ref[...]
    lb = lax.dot_general(h.astype(jnp.bfloat16), det_ref[...],
                         (((1,), (1,)), ((), ())),
                         preferred_element_type=jnp.float32)
    for g in range(_G):
        logits_ref[pl.ds(g * _BT * 128, _BT * 128)] = (
            lb[:, g * 128:(g + 1) * 128].reshape(_BT * 128))

    simts = []
    for g in range(2 * _H):
        qg = jnp.dot(h, wq_ref[:, g * _NK:(g + 1) * _NK],
                     preferred_element_type=jnp.float32)
        simts.append(lax.dot_general(kk_ref[g], qg, (((1,), (1,)), ((), ())),
                                     preferred_element_type=jnp.float32))

    riota = lax.broadcasted_iota(jnp.int32, (_NK, _BT), 0)
    ciota = lax.broadcasted_iota(jnp.int32, (_NCAND, _BT), 0)
    tloc = lax.broadcasted_iota(jnp.int32, (1, _BT), 1)
    row32 = lax.broadcasted_iota(jnp.int32, (4 * _K, _BT), 0)

    def top8(s):
        vals, poss = [], []
        for _ in range(_K):
            m = jnp.max(s, axis=0, keepdims=True)
            am = jnp.min(jnp.where(s == m, riota, _NK), axis=0, keepdims=True)
            vals.append(m)
            poss.append(am)
            s = jnp.where(riota == am, _NEG, s)
        return vals, poss

    out_ss = jnp.zeros((4 * _K, _BT), jnp.float32)
    out_gi = jnp.zeros((4 * _K, _BT), jnp.int32)
    for hh in range(_H):
        xv, xp = top8(simts[hh])
        yv, yp = top8(simts[_H + hh])
        asc = jnp.concatenate([xv[a] + yv[b] for a, b in _CAND], axis=0)
        aidx = jnp.concatenate([xp[a] * _NK + yp[b] for a, b in _CAND],
                               axis=0)
        scs, eids = [], []
        for _ in range(_K):
            m = jnp.max(asc, axis=0, keepdims=True)
            am = jnp.min(jnp.where(asc == m, ciota, _NCAND), axis=0,
                         keepdims=True)
            sel = ciota == am
            eids.append(jnp.sum(jnp.where(sel, aidx, 0), axis=0, keepdims=True))
            scs.append(m)
            asc = jnp.where(sel, _NEG, asc)
        es = [jnp.exp(v - scs[0]) for v in scs]
        tot = es[0]
        for e in es[1:]:
            tot = tot + e
        inv = 1.0 / tot
        for k in range(_K):
            r = hh * _K + k
            gf = (pid * (_G * _BT * 128) + (eids[k] // 128) * (_BT * 128)
                  + tloc * 128 + (eids[k] % 128))
            out_ss = jnp.where(row32 == r, es[k] * inv, out_ss)
            out_gi = jnp.where(row32 == r, gf, out_gi)
    ss_ref[...] = jnp.transpose(out_ss)
    gidx_ref[...] = jnp.transpose(out_gi)


def _combine_body(wsp_ref, ue_ref, out_ref):
    accs = [None] * 4
    for g in range(_G):
        d = jnp.dot(wsp_ref[g], ue_ref[g],
                    preferred_element_type=jnp.float32)
        a = g & 3
        accs[a] = d if accs[a] is None else accs[a] + d
    out_ref[...] = (accs[0] + accs[1]) + (accs[2] + accs[3])


def _sc_body(logits_hbm, gidx_hbm, ss_hbm, zc_hbm, wsp_hbm, gi0_v, gi1_v,
             ss_v, xg_v, wv_v, chunk_v, gsem, dsem0):
    cid = lax.axis_index("c")
    sid = lax.axis_index("s")
    wid = sid * _NC + cid
    base = wid * _TPW
    nsel = 4 * _K * _C

    zero16 = jnp.zeros((16,), jnp.float32)
    pltpu.sync_copy(zc_hbm, chunk_v)

    gis = [gi0_v, gi1_v]
    out_dma = None
    ptb = None
    for ci in range(_TPW // _C):
        b = ci & 1
        t0 = base + ci * _C
        tb = t0 & (_BT - 1)
        pltpu.sync_copy(gidx_hbm.at[pl.ds(t0 * 4 * _K, nsel)], gis[b])
        pltpu.sync_copy(ss_hbm.at[pl.ds(t0 * 4 * _K, nsel)], ss_v)
        pltpu.async_copy(logits_hbm.at[gis[b]], xg_v, gsem).wait()
        for j in range(nsel // 16):
            sl = pl.ds(j * 16, 16)
            z = xg_v[sl] * ss_v[sl]
            wv_v[sl] = z * (1.0 / (1.0 + jnp.exp(-z)))
        if out_dma is not None:
            out_dma.wait()
            gp = gis[1 - b]
            for j in range(nsel // 16):
                gf = gp[pl.ds(j * 16, 16)]
                plsc.store_scatter(
                    chunk_v,
                    [lax.shift_right_logical(gf, 15) & (_G - 1),
                     (lax.shift_right_logical(gf, 7) & (_BT - 1)) - ptb,
                     gf & 127],
                    zero16)
        for j in range(nsel // 16):
            sl = pl.ds(j * 16, 16)
            gf = gis[b][sl]
            plsc.addupdate_scatter(
                chunk_v,
                [lax.shift_right_logical(gf, 15) & (_G - 1),
                 (lax.shift_right_logical(gf, 7) & (_BT - 1)) - tb,
                 gf & 127],
                wv_v[sl])
        out_dma = pltpu.async_copy(chunk_v, wsp_hbm.at[:, pl.ds(t0, _C), :],
                                   dsem0)
        ptb = tb
    out_dma.wait()


def _routing_sc(logits, gidx, ss):
    mesh = plsc.VectorSubcoreMesh(core_axis_name="c", subcore_axis_name="s")
    f = pl.kernel(
        _sc_body,
        out_type=jax.ShapeDtypeStruct((_G, _T, 128), jnp.float32),
        mesh=mesh,
        scratch_types=[
            pltpu.VMEM((4 * _K * _C,), jnp.int32),
            pltpu.VMEM((4 * _K * _C,), jnp.int32),
            pltpu.VMEM((4 * _K * _C,), jnp.float32),
            pltpu.VMEM((4 * _K * _C,), jnp.float32),
            pltpu.VMEM((4 * _K * _C,), jnp.float32),
            pltpu.VMEM((_G, _C, 128), jnp.float32),
            pltpu.SemaphoreType.DMA,
            pltpu.SemaphoreType.DMA,
        ],
        compiler_params=pltpu.CompilerParams(needs_layout_passes=False),
    )
    return f(logits, gidx.reshape(_T * 4 * _K), ss.reshape(_T * 4 * _K),
             jnp.zeros((_G, _C, 128), jnp.float32))


def kernel(hidden_states, W_up, b_up, W_down, b_down, W_q, keys, down_embed,
           up_embed):
    x = hidden_states.reshape(_T, _DM)
    kk = keys.transpose(2, 0, 1, 3).reshape(2 * _H, _NK, _NK)

    grid = _T // _BT
    logits, gidx, ss = pl.pallas_call(
        _front_body,
        grid=(grid,),
        in_specs=[
            pl.BlockSpec((_BT, _DM), lambda i: (i, 0)),
            pl.BlockSpec((_DM, _DCD), lambda i: (0, 0)),
            pl.BlockSpec((1, _DCD), lambda i: (0, 0)),
            pl.BlockSpec((_DCD, _DPE), lambda i: (0, 0)),
            pl.BlockSpec((1, _DPE), lambda i: (0, 0)),
            pl.BlockSpec((_DPE, _DPE), lambda i: (0, 0)),
            pl.BlockSpec((2 * _H, _NK, _NK), lambda i: (0, 0, 0)),
            pl.BlockSpec((_NE, _DPE), lambda i: (0, 0)),
        ],
        out_specs=[
            pl.BlockSpec((_G * _BT * 128,), lambda i: (i,)),
            pl.BlockSpec((_BT, 4 * _K), lambda i: (i, 0)),
            pl.BlockSpec((_BT, 4 * _K), lambda i: (i, 0)),
        ],
        out_shape=[
            jax.ShapeDtypeStruct(((_T // _BT) * _G * _BT * 128,), jnp.float32),
            jax.ShapeDtypeStruct((_T, 4 * _K), jnp.int32),
            jax.ShapeDtypeStruct((_T, 4 * _K), jnp.float32),
        ],
    )(x, W_up, b_up.reshape(1, _DCD), W_down, b_down.reshape(1, _DPE), W_q,
      kk, down_embed.astype(jnp.bfloat16))

    wsp = _routing_sc(logits, gidx, ss)

    ueb = up_embed.reshape(_G, 128, _DM)
    out = pl.pallas_call(
        _combine_body,
        grid=(grid,),
        in_specs=[
            pl.BlockSpec((_G, _BT, 128), lambda i: (0, i, 0)),
            pl.BlockSpec((_G, 128, _DM), lambda i: (0, 0, 0)),
        ],
        out_specs=pl.BlockSpec((_BT, _DM), lambda i: (i, 0)),
        out_shape=jax.ShapeDtypeStruct((_T, _DM), jnp.float32),
    )(wsp, ueb)

    return out.reshape(1, _T, _DM)
